# baseline (speedup 1.0000x reference)
import functools

import jax
import jax.numpy as jnp
from jax import lax
from jax.experimental import pallas as pl
from jax.experimental.pallas import tpu as pltpu
from jax.experimental.pallas import tpu_sc as plsc

N = 18432
D = 64
DP = 128
K = 1024
R = 2048
NB = N // R

_NC = 2
_NS = 16
NW = _NC * _NS
BPW = N // NW


def _chain_sum_64(sq, axis):
    if axis == 1:
        a = sq[:, 0:8]
        for t in range(1, 8):
            a = a + sq[:, 8 * t:8 * (t + 1)]
        p = a[:, 0:4] + a[:, 4:8]
        return (p[:, 0:1] + p[:, 2:3]) + (p[:, 1:2] + p[:, 3:4])
    a = sq[0:8, :]
    for t in range(1, 8):
        a = a + sq[8 * t:8 * (t + 1), :]
    p = a[0:4, :] + a[4:8, :]
    return (p[0:1, :] + p[2:3, :]) + (p[1:2, :] + p[3:4, :])


def _tc_body(x_ref, emb_ref, enc_ref, idx_ref, tbl_ref):
    x = x_ref[...]
    emb = emb_ref[...]
    sim = jnp.dot(x, emb, preferred_element_type=jnp.float32)
    x2 = _chain_sum_64(x * x, axis=1)
    e2 = _chain_sum_64(emb * emb, axis=0)
    scores = (x2 + e2) - 2.0 * sim
    m = jnp.min(scores, axis=1, keepdims=True)
    cols = lax.broadcasted_iota(jnp.int32, (R, K), 1)
    idx = jnp.min(jnp.where(scores == m, cols, K), axis=1).astype(jnp.int32)
    enc_ref[...] = jnp.where(cols == idx[:, None], 1.0, 0.0)
    idx_ref[...] = idx

    @pl.when(pl.program_id(0) == 0)
    def _write_table():
        tbl_ref[...] = jnp.concatenate(
            [emb.T, jnp.zeros((K, DP - D), jnp.float32)], axis=1)


_tc_call = pl.pallas_call(
    _tc_body,
    grid=(NB,),
    in_specs=[
        pl.BlockSpec((R, D), lambda i: (i, 0)),
        pl.BlockSpec((D, K), lambda i: (0, 0)),
    ],
    out_specs=[
        pl.BlockSpec((R, K), lambda i: (i, 0)),
        pl.BlockSpec((R,), lambda i: (i,)),
        pl.BlockSpec((K, DP), lambda i: (0, 0)),
    ],
    out_shape=[
        jax.ShapeDtypeStruct((N, K), jnp.float32),
        jax.ShapeDtypeStruct((N,), jnp.int32),
        jax.ShapeDtypeStruct((K, DP), jnp.float32),
    ],
    compiler_params=pltpu.CompilerParams(
        dimension_semantics=("arbitrary",),
    ),
)


@functools.cache
def _make_sc_gather():
    mesh = plsc.VectorSubcoreMesh(
        core_axis_name="c", subcore_axis_name="s", num_cores=_NC)

    @functools.partial(
        pl.kernel,
        mesh=mesh,
        out_type=jax.ShapeDtypeStruct((N, DP), jnp.float32),
        scratch_types=[
            pltpu.VMEM((BPW,), jnp.int32),
            pltpu.VMEM((BPW, DP), jnp.float32),
            pltpu.SemaphoreType.DMA,
        ],
    )
    def _sc_gather(table_hbm, idx_hbm, out_hbm, idx_v, rows_v, sem):
        wid = lax.axis_index("s") * _NC + lax.axis_index("c")
        base = wid * BPW
        pltpu.sync_copy(idx_hbm.at[pl.ds(base, BPW)], idx_v)
        pltpu.async_copy(table_hbm.at[idx_v], rows_v, sem).wait()
        pltpu.sync_copy(rows_v, out_hbm.at[pl.ds(base, BPW)])

    return _sc_gather


def kernel(x, embeddings):
    encodings, idx, table = _tc_call(x, embeddings)
    quantized_padded = _make_sc_gather()(table, idx)
    return (encodings, quantized_padded[:, :D])


# speedup vs baseline: 1.3551x; 1.3551x over previous
import functools

import jax
import jax.numpy as jnp
from jax import lax
from jax.experimental import pallas as pl
from jax.experimental.pallas import tpu as pltpu
from jax.experimental.pallas import tpu_sc as plsc

N = 18432
D = 64
DP = 128
K = 1024
R = 2048
NB = N // R

_NC = 2
_NS = 16
NW = _NC * _NS
BPW = N // NW


def _chain_sum_64(sq, axis):
    if axis == 1:
        a = sq[:, 0:8]
        for t in range(1, 8):
            a = a + sq[:, 8 * t:8 * (t + 1)]
        p = a[:, 0:4] + a[:, 4:8]
        return (p[:, 0:1] + p[:, 2:3]) + (p[:, 1:2] + p[:, 3:4])
    a = sq[0:8, :]
    for t in range(1, 8):
        a = a + sq[8 * t:8 * (t + 1), :]
    p = a[0:4, :] + a[4:8, :]
    return (p[0:1, :] + p[2:3, :]) + (p[1:2, :] + p[3:4, :])


def _tc_body(xt_ref, emb_ref, enc_ref, idx_ref, tbl_ref):
    xt = xt_ref[...]
    emb = emb_ref[...]
    sim = lax.dot_general(
        xt, emb, (((0,), (0,)), ((), ())),
        preferred_element_type=jnp.float32)
    x2 = _chain_sum_64(xt * xt, axis=0).T
    e2 = _chain_sum_64(emb * emb, axis=0)
    scores = (x2 + e2) - 2.0 * sim
    m = jnp.min(scores, axis=1, keepdims=True)
    cols = lax.broadcasted_iota(jnp.int32, (R, K), 1)
    idx = jnp.min(jnp.where(scores == m, cols, K), axis=1).astype(jnp.int32)
    enc_ref[...] = jnp.where(cols == idx[:, None], 1.0, 0.0)
    idx_ref[...] = idx

    @pl.when(pl.program_id(0) == 0)
    def _write_table():
        tbl_ref[...] = jnp.concatenate(
            [emb.T, jnp.zeros((K, DP - D), jnp.float32)], axis=1)


_tc_call = pl.pallas_call(
    _tc_body,
    grid=(NB,),
    in_specs=[
        pl.BlockSpec((D, R), lambda i: (0, i)),
        pl.BlockSpec((D, K), lambda i: (0, 0)),
    ],
    out_specs=[
        pl.BlockSpec((R, K), lambda i: (i, 0)),
        pl.BlockSpec((R,), lambda i: (i,)),
        pl.BlockSpec((K, DP), lambda i: (0, 0)),
    ],
    out_shape=[
        jax.ShapeDtypeStruct((N, K), jnp.float32),
        jax.ShapeDtypeStruct((N,), jnp.int32),
        jax.ShapeDtypeStruct((K, DP), jnp.float32),
    ],
    compiler_params=pltpu.CompilerParams(
        dimension_semantics=("arbitrary",),
    ),
)


@functools.cache
def _make_sc_gather():
    mesh = plsc.VectorSubcoreMesh(
        core_axis_name="c", subcore_axis_name="s", num_cores=_NC)

    @functools.partial(
        pl.kernel,
        mesh=mesh,
        out_type=jax.ShapeDtypeStruct((N, DP), jnp.float32),
        scratch_types=[
            pltpu.VMEM((BPW,), jnp.int32),
            pltpu.VMEM((BPW, DP), jnp.float32),
            pltpu.SemaphoreType.DMA,
        ],
    )
    def _sc_gather(table_hbm, idx_hbm, out_hbm, idx_v, rows_v, sem):
        wid = lax.axis_index("s") * _NC + lax.axis_index("c")
        base = wid * BPW
        pltpu.sync_copy(idx_hbm.at[pl.ds(base, BPW)], idx_v)
        pltpu.async_copy(table_hbm.at[idx_v], rows_v, sem).wait()
        pltpu.sync_copy(rows_v, out_hbm.at[pl.ds(base, BPW)])

    return _sc_gather


def kernel(x, embeddings):
    encodings, idx, table = _tc_call(x.T, embeddings)
    quantized_padded = _make_sc_gather()(table, idx)
    return (encodings, quantized_padded[:, :D])
